# baseline (speedup 1.0000x reference)
import functools

import jax
import jax.numpy as jnp
from jax import lax
from jax.experimental import pallas as pl
from jax.experimental.pallas import tpu as pltpu
from jax.experimental.pallas import tpu_sc as plsc


def _sc_gather(W, tok, NC, NS, L):
    P, V = W.shape
    N = tok.shape[0]
    NW = NC * NS
    ROWS = P // NW
    CH = 2048

    mesh = plsc.VectorSubcoreMesh(core_axis_name="c", subcore_axis_name="s")

    @functools.partial(
        pl.kernel,
        mesh=mesh,
        compiler_params=pltpu.CompilerParams(needs_layout_passes=False),
        out_type=jax.ShapeDtypeStruct((P, N), jnp.float32),
        scratch_types=[
            pltpu.VMEM((V,), jnp.float32),
            pltpu.VMEM((N,), jnp.int32),
            pltpu.VMEM((CH,), jnp.float32),
        ],
    )
    def sc_kernel(w_hbm, tok_hbm, ut_hbm, wrow, idxs, outb):
        wid = lax.axis_index("s") * NC + lax.axis_index("c")
        pltpu.sync_copy(tok_hbm, idxs)
        for r in range(ROWS):
            p = wid * ROWS + r
            pltpu.sync_copy(w_hbm.at[p], wrow)

            def chunk_body(c, carry):
                base = c * CH

                def vec_body(j, carry2):
                    off = j * L
                    idx = idxs[pl.ds(base + off, L)]
                    outb[pl.ds(off, L)] = plsc.load_gather(wrow, [idx])
                    return carry2

                lax.fori_loop(0, CH // L, vec_body, 0, unroll=4)
                pltpu.sync_copy(outb, ut_hbm.at[p, pl.ds(base, CH)])
                return carry

            lax.fori_loop(0, N // CH, chunk_body, 0)

    return sc_kernel(W, tok)


def _tc_expand(ut, fvec, spr, BN):
    P, N = ut.shape
    PA = fvec.shape[1]
    A = PA // P

    def tc_body(ut_ref, fvec_ref, sp_ref, u_ref, fh_ref, y_ref):
        u = ut_ref[...].T
        fh = (u >= sp_ref[...]).astype(jnp.float32)
        t = fh * u
        col = lax.broadcasted_iota(jnp.int32, (P, PA), 1)
        row = lax.broadcasted_iota(jnp.int32, (P, PA), 0)
        E = jnp.where(
            (col // A) == row, jnp.broadcast_to(fvec_ref[...], (P, PA)), 0.0
        )
        y_ref[...] = jax.lax.dot(
            t, E, precision=jax.lax.Precision.DEFAULT,
            preferred_element_type=jnp.float32,
        )
        u_ref[...] = u
        fh_ref[...] = fh

    return pl.pallas_call(
        tc_body,
        grid=(N // BN,),
        in_specs=[
            pl.BlockSpec((P, BN), lambda i: (0, i)),
            pl.BlockSpec((1, PA), lambda i: (0, 0)),
            pl.BlockSpec((1, P), lambda i: (0, 0)),
        ],
        out_specs=[
            pl.BlockSpec((BN, P), lambda i: (i, 0)),
            pl.BlockSpec((BN, P), lambda i: (i, 0)),
            pl.BlockSpec((BN, PA), lambda i: (i, 0)),
        ],
        out_shape=[
            jax.ShapeDtypeStruct((N, P), jnp.float32),
            jax.ShapeDtypeStruct((N, P), jnp.float32),
            jax.ShapeDtypeStruct((N, PA), jnp.float32),
        ],
    )(ut, fvec, spr)


def kernel(input_ids, W, f, SP):
    P, V = W.shape
    A = f.shape[1]
    tok = input_ids.reshape(-1)
    N = tok.shape[0]

    info = plsc.get_sparse_core_info()
    ut = _sc_gather(W, tok, info.num_cores, info.num_subcores, info.num_lanes)

    fvec = f.reshape(1, P * A)
    spr = SP.reshape(1, P)
    u, fh, y2 = _tc_expand(ut, fvec, spr, BN=512)
    return u, fh, y2.reshape(N, P, A)


# speedup vs baseline: 1.0539x; 1.0539x over previous
import functools

import jax
import jax.numpy as jnp
from jax import lax
from jax.experimental import pallas as pl
from jax.experimental.pallas import tpu as pltpu
from jax.experimental.pallas import tpu_sc as plsc


def _sc_gather(W, tok, NC, NS, L):
    P, V = W.shape
    N = tok.shape[0]
    NW = NC * NS
    ROWS = P // NW
    CH = 2048

    mesh = plsc.VectorSubcoreMesh(core_axis_name="c", subcore_axis_name="s")

    @functools.partial(
        pl.kernel,
        mesh=mesh,
        compiler_params=pltpu.CompilerParams(needs_layout_passes=False),
        out_type=jax.ShapeDtypeStruct((P, N), jnp.float32),
        scratch_types=[
            pltpu.VMEM((V,), jnp.float32),
            pltpu.VMEM((N,), jnp.int32),
            pltpu.VMEM((CH,), jnp.float32),
        ],
    )
    def sc_kernel(w_hbm, tok_hbm, ut_hbm, wrow, idxs, outb):
        wid = lax.axis_index("s") * NC + lax.axis_index("c")
        pltpu.sync_copy(tok_hbm, idxs)
        for r in range(ROWS):
            p = wid * ROWS + r
            pltpu.sync_copy(w_hbm.at[p], wrow)

            def chunk_body(c, carry):
                base = c * CH

                def vec_body(j, carry2):
                    off = j * L
                    idx = idxs[pl.ds(base + off, L)]
                    outb[pl.ds(off, L)] = plsc.load_gather(wrow, [idx])
                    return carry2

                lax.fori_loop(0, CH // L, vec_body, 0, unroll=4)
                pltpu.sync_copy(outb, ut_hbm.at[p, pl.ds(base, CH)])
                return carry

            lax.fori_loop(0, N // CH, chunk_body, 0)

    return sc_kernel(W, tok)


def _tc_expand(ut, fvec, spr, BN):
    P, N = ut.shape
    PA = fvec.shape[1]
    A = PA // P

    def tc_body(ut_ref, fvec_ref, sp_ref, u_ref, fh_ref, y_ref, e_ref):
        @pl.when(pl.program_id(0) == 0)
        def _build_e():
            col = lax.broadcasted_iota(jnp.int32, (P, PA), 1)
            row = lax.broadcasted_iota(jnp.int32, (P, PA), 0)
            e_ref[...] = jnp.where(
                (col // A) == row,
                jnp.broadcast_to(fvec_ref[...], (P, PA)),
                0.0,
            )

        u = ut_ref[...].T
        fh = (u >= sp_ref[...]).astype(jnp.float32)
        t = fh * u
        y_ref[...] = jax.lax.dot(
            t, e_ref[...], precision=jax.lax.Precision.DEFAULT,
            preferred_element_type=jnp.float32,
        )
        u_ref[...] = u
        fh_ref[...] = fh

    return pl.pallas_call(
        tc_body,
        grid=(N // BN,),
        in_specs=[
            pl.BlockSpec((P, BN), lambda i: (0, i)),
            pl.BlockSpec((1, PA), lambda i: (0, 0)),
            pl.BlockSpec((1, P), lambda i: (0, 0)),
        ],
        out_specs=[
            pl.BlockSpec((BN, P), lambda i: (i, 0)),
            pl.BlockSpec((BN, P), lambda i: (i, 0)),
            pl.BlockSpec((BN, PA), lambda i: (i, 0)),
        ],
        out_shape=[
            jax.ShapeDtypeStruct((N, P), jnp.float32),
            jax.ShapeDtypeStruct((N, P), jnp.float32),
            jax.ShapeDtypeStruct((N, PA), jnp.float32),
        ],
        scratch_shapes=[pltpu.VMEM((P, PA), jnp.float32)],
    )(ut, fvec, spr)


def kernel(input_ids, W, f, SP):
    P, V = W.shape
    A = f.shape[1]
    tok = input_ids.reshape(-1)
    N = tok.shape[0]

    info = plsc.get_sparse_core_info()
    ut = _sc_gather(W, tok, info.num_cores, info.num_subcores, info.num_lanes)

    fvec = f.reshape(1, P * A)
    spr = SP.reshape(1, P)
    u, fh, y2 = _tc_expand(ut, fvec, spr, BN=1024)
    return u, fh, y2.reshape(N, P, A)


# speedup vs baseline: 1.0812x; 1.0260x over previous
import functools

import jax
import jax.numpy as jnp
from jax import lax
from jax.experimental import pallas as pl
from jax.experimental.pallas import tpu as pltpu
from jax.experimental.pallas import tpu_sc as plsc


def _sc_gather(W, tok, NC, NS, L):
    P, V = W.shape
    N = tok.shape[0]
    NW = NC * NS
    ROWS = P // NW
    CH = 2048

    mesh = plsc.VectorSubcoreMesh(core_axis_name="c", subcore_axis_name="s")

    NCH = N // CH
    assert NCH % 2 == 0

    @functools.partial(
        pl.kernel,
        mesh=mesh,
        compiler_params=pltpu.CompilerParams(needs_layout_passes=False),
        out_type=jax.ShapeDtypeStruct((P, N), jnp.float32),
        scratch_types=[
            pltpu.VMEM((V,), jnp.float32),
            pltpu.VMEM((N,), jnp.int32),
            pltpu.VMEM((CH,), jnp.float32),
            pltpu.VMEM((CH,), jnp.float32),
            pltpu.SemaphoreType.DMA,
            pltpu.SemaphoreType.DMA,
        ],
    )
    def sc_kernel(w_hbm, tok_hbm, ut_hbm, wrow, idxs, ob0, ob1, s0, s1):
        wid = lax.axis_index("s") * NC + lax.axis_index("c")
        pltpu.sync_copy(tok_hbm, idxs)
        for r in range(ROWS):
            p = wid * ROWS + r
            pltpu.sync_copy(w_hbm.at[p], wrow)

            def gather_chunk(base, ob):
                def vec_body(j, carry):
                    off = j * L
                    idx = idxs[pl.ds(base + off, L)]
                    ob[pl.ds(off, L)] = plsc.load_gather(wrow, [idx])
                    return carry

                lax.fori_loop(0, CH // L, vec_body, 0, unroll=8)

            def pair_body(k, carry):
                for b, (ob, sem) in enumerate(((ob0, s0), (ob1, s1))):
                    c = k * 2 + b

                    @pl.when(k > 0)
                    def _wait_prev():
                        pltpu.make_async_copy(
                            ob, ut_hbm.at[p, pl.ds(0, CH)], sem
                        ).wait()

                    gather_chunk(c * CH, ob)
                    pltpu.async_copy(ob, ut_hbm.at[p, pl.ds(c * CH, CH)], sem)
                return carry

            lax.fori_loop(0, NCH // 2, pair_body, 0)
            for ob, sem in ((ob0, s0), (ob1, s1)):
                pltpu.make_async_copy(ob, ut_hbm.at[p, pl.ds(0, CH)], sem).wait()

    return sc_kernel(W, tok)


def _tc_expand(ut, fvec, spr, BN):
    P, N = ut.shape
    PA = fvec.shape[1]
    A = PA // P

    def tc_body(ut_ref, fvec_ref, sp_ref, u_ref, fh_ref, y_ref, e_ref):
        @pl.when(pl.program_id(0) == 0)
        def _build_e():
            col = lax.broadcasted_iota(jnp.int32, (P, PA), 1)
            row = lax.broadcasted_iota(jnp.int32, (P, PA), 0)
            e_ref[...] = jnp.where(
                (col // A) == row,
                jnp.broadcast_to(fvec_ref[...], (P, PA)),
                0.0,
            )

        u = ut_ref[...].T
        fh = (u >= sp_ref[...]).astype(jnp.float32)
        t = fh * u
        y_ref[...] = jax.lax.dot(
            t, e_ref[...], precision=jax.lax.Precision.DEFAULT,
            preferred_element_type=jnp.float32,
        )
        u_ref[...] = u
        fh_ref[...] = fh

    return pl.pallas_call(
        tc_body,
        grid=(N // BN,),
        in_specs=[
            pl.BlockSpec((P, BN), lambda i: (0, i)),
            pl.BlockSpec((1, PA), lambda i: (0, 0)),
            pl.BlockSpec((1, P), lambda i: (0, 0)),
        ],
        out_specs=[
            pl.BlockSpec((BN, P), lambda i: (i, 0)),
            pl.BlockSpec((BN, P), lambda i: (i, 0)),
            pl.BlockSpec((BN, PA), lambda i: (i, 0)),
        ],
        out_shape=[
            jax.ShapeDtypeStruct((N, P), jnp.float32),
            jax.ShapeDtypeStruct((N, P), jnp.float32),
            jax.ShapeDtypeStruct((N, PA), jnp.float32),
        ],
        scratch_shapes=[pltpu.VMEM((P, PA), jnp.float32)],
    )(ut, fvec, spr)


def kernel(input_ids, W, f, SP):
    P, V = W.shape
    A = f.shape[1]
    tok = input_ids.reshape(-1)
    N = tok.shape[0]

    info = plsc.get_sparse_core_info()
    ut = _sc_gather(W, tok, info.num_cores, info.num_subcores, info.num_lanes)

    fvec = f.reshape(1, P * A)
    spr = SP.reshape(1, P)
    u, fh, y2 = _tc_expand(ut, fvec, spr, BN=1024)
    return u, fh, y2.reshape(N, P, A)


# speedup vs baseline: 1.1612x; 1.0740x over previous
import functools

import jax
import jax.numpy as jnp
from jax import lax
from jax.experimental import pallas as pl
from jax.experimental.pallas import tpu as pltpu
from jax.experimental.pallas import tpu_sc as plsc


def _sc_gather(W, tok, SP, NC, NS, L):
    P, V = W.shape
    N = tok.shape[0]
    NW = NC * NS
    ROWS = P // NW
    CH = 1280
    NCH = N // CH
    assert NCH % 2 == 0

    mesh = plsc.VectorSubcoreMesh(core_axis_name="c", subcore_axis_name="s")
    row_t = jax.ShapeDtypeStruct((P, N), jnp.float32)

    @functools.partial(
        pl.kernel,
        mesh=mesh,
        compiler_params=pltpu.CompilerParams(needs_layout_passes=False),
        out_type=(row_t, row_t, row_t),
        scratch_types=[
            pltpu.VMEM((V,), jnp.float32),
            pltpu.VMEM((N,), jnp.int32),
            pltpu.VMEM((P,), jnp.float32),
            pltpu.VMEM((CH,), jnp.float32),
            pltpu.VMEM((CH,), jnp.float32),
            pltpu.VMEM((CH,), jnp.float32),
            pltpu.VMEM((CH,), jnp.float32),
            pltpu.VMEM((CH,), jnp.float32),
            pltpu.VMEM((CH,), jnp.float32),
            pltpu.SemaphoreType.DMA,
            pltpu.SemaphoreType.DMA,
            pltpu.SemaphoreType.DMA,
            pltpu.SemaphoreType.DMA,
            pltpu.SemaphoreType.DMA,
            pltpu.SemaphoreType.DMA,
        ],
    )
    def sc_kernel(w_hbm, tok_hbm, sp_hbm, ut_hbm, ft_hbm, tt_hbm,
                  wrow, idxs, sp_v, ub0, ub1, fb0, fb1, tb0, tb1,
                  su0, su1, sf0, sf1, st0, st1):
        wid = lax.axis_index("s") * NC + lax.axis_index("c")
        pltpu.sync_copy(tok_hbm, idxs)
        pltpu.sync_copy(sp_hbm, sp_v)
        bufs = ((ub0, fb0, tb0, su0, sf0, st0),
                (ub1, fb1, tb1, su1, sf1, st1))
        for r in range(ROWS):
            p = wid * ROWS + r
            sp16 = plsc.load_gather(sp_v, [jnp.full((L,), p, jnp.int32)])
            pltpu.sync_copy(w_hbm.at[p], wrow)

            def pair_body(k, carry, sp16=sp16, p=p):
                for b, (ub, fb, tb, su, sf, st) in enumerate(bufs):
                    c = k * 2 + b

                    @pl.when(k > 0)
                    def _wait_prev():
                        for buf, sem in ((ub, su), (fb, sf), (tb, st)):
                            pltpu.make_async_copy(
                                buf, ut_hbm.at[p, pl.ds(0, CH)], sem
                            ).wait()

                    base = c * CH

                    def vec_body(j, carry2):
                        off = j * L
                        idx = idxs[pl.ds(base + off, L)]
                        u16 = plsc.load_gather(wrow, [idx])
                        f16 = jnp.where(
                            u16 >= sp16, jnp.float32(1.0), jnp.float32(0.0)
                        )
                        ub[pl.ds(off, L)] = u16
                        fb[pl.ds(off, L)] = f16
                        tb[pl.ds(off, L)] = f16 * u16
                        return carry2

                    lax.fori_loop(0, CH // L, vec_body, 0, unroll=8)
                    pltpu.async_copy(ub, ut_hbm.at[p, pl.ds(base, CH)], su)
                    pltpu.async_copy(fb, ft_hbm.at[p, pl.ds(base, CH)], sf)
                    pltpu.async_copy(tb, tt_hbm.at[p, pl.ds(base, CH)], st)
                return carry

            lax.fori_loop(0, NCH // 2, pair_body, 0)
            for ub, fb, tb, su, sf, st in bufs:
                for buf, sem in ((ub, su), (fb, sf), (tb, st)):
                    pltpu.make_async_copy(
                        buf, ut_hbm.at[p, pl.ds(0, CH)], sem
                    ).wait()

    return sc_kernel(W, tok, SP)


def _tc_expand(tt, fvec, BN):
    P, N = tt.shape
    PA = fvec.shape[1]
    A = PA // P

    def tc_body(tt_ref, fvec_ref, y_ref, e_ref):
        @pl.when(pl.program_id(0) == 0)
        def _build_e():
            col = lax.broadcasted_iota(jnp.int32, (P, PA), 1)
            row = lax.broadcasted_iota(jnp.int32, (P, PA), 0)
            e_ref[...] = jnp.where(
                (col // A) == row,
                jnp.broadcast_to(fvec_ref[...], (P, PA)),
                0.0,
            )

        y_ref[...] = jax.lax.dot_general(
            tt_ref[...], e_ref[...],
            (((0,), (0,)), ((), ())),
            precision=jax.lax.Precision.DEFAULT,
            preferred_element_type=jnp.float32,
        )

    return pl.pallas_call(
        tc_body,
        grid=(N // BN,),
        in_specs=[
            pl.BlockSpec((P, BN), lambda i: (0, i)),
            pl.BlockSpec((1, PA), lambda i: (0, 0)),
        ],
        out_specs=pl.BlockSpec((BN, PA), lambda i: (i, 0)),
        out_shape=jax.ShapeDtypeStruct((N, PA), jnp.float32),
        scratch_shapes=[pltpu.VMEM((P, PA), jnp.float32)],
    )(tt, fvec)


def kernel(input_ids, W, f, SP):
    P, V = W.shape
    A = f.shape[1]
    tok = input_ids.reshape(-1)
    N = tok.shape[0]

    info = plsc.get_sparse_core_info()
    ut, ft, tt = _sc_gather(
        W, tok, SP, info.num_cores, info.num_subcores, info.num_lanes
    )

    fvec = f.reshape(1, P * A)
    y2 = _tc_expand(tt, fvec, BN=1024)
    return ut.T, ft.T, y2.reshape(N, P, A)


# speedup vs baseline: 2.0107x; 1.7316x over previous
import functools

import jax
import jax.numpy as jnp
from jax import lax
from jax.experimental import pallas as pl
from jax.experimental.pallas import tpu as pltpu
from jax.experimental.pallas import tpu_sc as plsc


def _sc_gather(W, tok, SP, NC, NS, L):
    P, V = W.shape
    N = tok.shape[0]
    NW = NC * NS
    ROWS = P // NW
    CH = 1280
    NCH = N // CH
    assert NCH % 2 == 0

    mesh = plsc.VectorSubcoreMesh(core_axis_name="c", subcore_axis_name="s")
    row_t = jax.ShapeDtypeStruct((P, N), jnp.float32)

    @functools.partial(
        pl.kernel,
        mesh=mesh,
        compiler_params=pltpu.CompilerParams(needs_layout_passes=False),
        out_type=(row_t, row_t, row_t),
        scratch_types=[
            pltpu.VMEM((V,), jnp.float32),
            pltpu.VMEM((N,), jnp.int32),
            pltpu.VMEM((P,), jnp.float32),
            pltpu.VMEM((CH,), jnp.float32),
            pltpu.VMEM((CH,), jnp.float32),
            pltpu.VMEM((CH,), jnp.float32),
            pltpu.VMEM((CH,), jnp.float32),
            pltpu.VMEM((CH,), jnp.float32),
            pltpu.VMEM((CH,), jnp.float32),
            pltpu.SemaphoreType.DMA,
            pltpu.SemaphoreType.DMA,
            pltpu.SemaphoreType.DMA,
            pltpu.SemaphoreType.DMA,
            pltpu.SemaphoreType.DMA,
            pltpu.SemaphoreType.DMA,
        ],
    )
    def sc_kernel(w_hbm, tok_hbm, sp_hbm, ut_hbm, ft_hbm, tt_hbm,
                  wrow, idxs, sp_v, ub0, ub1, fb0, fb1, tb0, tb1,
                  su0, su1, sf0, sf1, st0, st1):
        wid = lax.axis_index("s") * NC + lax.axis_index("c")
        pltpu.sync_copy(tok_hbm, idxs)
        pltpu.sync_copy(sp_hbm, sp_v)
        bufs = ((ub0, fb0, tb0, su0, sf0, st0),
                (ub1, fb1, tb1, su1, sf1, st1))
        for r in range(ROWS):
            p = wid * ROWS + r
            sp16 = plsc.load_gather(sp_v, [jnp.full((L,), p, jnp.int32)])
            pltpu.sync_copy(w_hbm.at[p], wrow)

            def pair_body(k, carry, sp16=sp16, p=p):
                for b, (ub, fb, tb, su, sf, st) in enumerate(bufs):
                    c = k * 2 + b

                    @pl.when(k > 0)
                    def _wait_prev():
                        for buf, sem in ((ub, su), (fb, sf), (tb, st)):
                            pltpu.make_async_copy(
                                buf, ut_hbm.at[p, pl.ds(0, CH)], sem
                            ).wait()

                    base = c * CH

                    def vec_body(j, carry2):
                        off = j * L
                        idx = idxs[pl.ds(base + off, L)]
                        u16 = plsc.load_gather(wrow, [idx])
                        f16 = jnp.where(
                            u16 >= sp16, jnp.float32(1.0), jnp.float32(0.0)
                        )
                        ub[pl.ds(off, L)] = u16
                        fb[pl.ds(off, L)] = f16
                        tb[pl.ds(off, L)] = f16 * u16
                        return carry2

                    lax.fori_loop(0, CH // L, vec_body, 0, unroll=8)
                    pltpu.async_copy(ub, ut_hbm.at[p, pl.ds(base, CH)], su)
                    pltpu.async_copy(fb, ft_hbm.at[p, pl.ds(base, CH)], sf)
                    pltpu.async_copy(tb, tt_hbm.at[p, pl.ds(base, CH)], st)
                return carry

            lax.fori_loop(0, NCH // 2, pair_body, 0)
            for ub, fb, tb, su, sf, st in bufs:
                for buf, sem in ((ub, su), (fb, sf), (tb, st)):
                    pltpu.make_async_copy(
                        buf, ut_hbm.at[p, pl.ds(0, CH)], sem
                    ).wait()

    return sc_kernel(W, tok, SP)


def _tc_expand(tt, fvec, BN):
    P, N = tt.shape
    PA = fvec.shape[1]
    A = PA // P

    def tc_body(tt_ref, fvec_ref, y_ref, e_ref):
        @pl.when(pl.program_id(0) == 0)
        def _build_e():
            col = lax.broadcasted_iota(jnp.int32, (P, PA), 1)
            row = lax.broadcasted_iota(jnp.int32, (P, PA), 0)
            e_ref[...] = jnp.where(
                (col // A) == row,
                jnp.broadcast_to(fvec_ref[...], (P, PA)),
                0.0,
            )

        y_ref[...] = jax.lax.dot_general(
            e_ref[...], tt_ref[...],
            (((0,), (0,)), ((), ())),
            precision=jax.lax.Precision.DEFAULT,
            preferred_element_type=jnp.float32,
        )

    return pl.pallas_call(
        tc_body,
        grid=(N // BN,),
        in_specs=[
            pl.BlockSpec((P, BN), lambda i: (0, i)),
            pl.BlockSpec((1, PA), lambda i: (0, 0)),
        ],
        out_specs=pl.BlockSpec((PA, BN), lambda i: (0, i)),
        out_shape=jax.ShapeDtypeStruct((PA, N), jnp.float32),
        scratch_shapes=[pltpu.VMEM((P, PA), jnp.float32)],
    )(tt, fvec)


def kernel(input_ids, W, f, SP):
    P, V = W.shape
    A = f.shape[1]
    tok = input_ids.reshape(-1)
    N = tok.shape[0]

    info = plsc.get_sparse_core_info()
    ut, ft, tt = _sc_gather(
        W, tok, SP, info.num_cores, info.num_subcores, info.num_lanes
    )

    fvec = f.reshape(1, P * A)
    z = _tc_expand(tt, fvec, BN=1024)
    y = z.reshape(P, A, N).transpose(2, 0, 1)
    return ut.T, ft.T, y


# speedup vs baseline: 2.1588x; 1.0736x over previous
import functools

import jax
import jax.numpy as jnp
from jax import lax
from jax.experimental import pallas as pl
from jax.experimental.pallas import tpu as pltpu
from jax.experimental.pallas import tpu_sc as plsc


def _sc_gather(W, tok, NC, NS, L):
    P, V = W.shape
    N = tok.shape[0]
    NW = NC * NS
    ROWS = P // NW
    CH = 2048
    NCH = N // CH
    assert NCH % 2 == 0

    mesh = plsc.VectorSubcoreMesh(core_axis_name="c", subcore_axis_name="s")

    @functools.partial(
        pl.kernel,
        mesh=mesh,
        compiler_params=pltpu.CompilerParams(needs_layout_passes=False),
        out_type=jax.ShapeDtypeStruct((P, N), jnp.float32),
        scratch_types=[
            pltpu.VMEM((V,), jnp.float32),
            pltpu.VMEM((N,), jnp.int32),
            pltpu.VMEM((CH,), jnp.float32),
            pltpu.VMEM((CH,), jnp.float32),
            pltpu.SemaphoreType.DMA,
            pltpu.SemaphoreType.DMA,
        ],
    )
    def sc_kernel(w_hbm, tok_hbm, ut_hbm, wrow, idxs, ub0, ub1, su0, su1):
        wid = lax.axis_index("s") * NC + lax.axis_index("c")
        pltpu.sync_copy(tok_hbm, idxs)
        bufs = ((ub0, su0), (ub1, su1))
        for r in range(ROWS):
            p = wid * ROWS + r
            pltpu.sync_copy(w_hbm.at[p], wrow)

            def pair_body(k, carry, p=p):
                for b, (ub, su) in enumerate(bufs):
                    c = k * 2 + b

                    @pl.when(k > 0)
                    def _wait_prev():
                        pltpu.make_async_copy(
                            ub, ut_hbm.at[p, pl.ds(0, CH)], su
                        ).wait()

                    base = c * CH

                    def vec_body(j, carry2):
                        off = j * L
                        idx = idxs[pl.ds(base + off, L)]
                        ub[pl.ds(off, L)] = plsc.load_gather(wrow, [idx])
                        return carry2

                    lax.fori_loop(0, CH // L, vec_body, 0, unroll=16)
                    pltpu.async_copy(ub, ut_hbm.at[p, pl.ds(base, CH)], su)
                return carry

            lax.fori_loop(0, NCH // 2, pair_body, 0)
            for ub, su in bufs:
                pltpu.make_async_copy(
                    ub, ut_hbm.at[p, pl.ds(0, CH)], su
                ).wait()

    return sc_kernel(W, tok)


def _tc_expand(ut, fvec, spc, BN):
    P, N = ut.shape
    PA = fvec.shape[1]
    A = PA // P

    def tc_body(ut_ref, fvec_ref, spc_ref, ft_ref, y_ref, e_ref):
        @pl.when(pl.program_id(0) == 0)
        def _build_e():
            col = lax.broadcasted_iota(jnp.int32, (P, PA), 1)
            row = lax.broadcasted_iota(jnp.int32, (P, PA), 0)
            e_ref[...] = jnp.where(
                (col // A) == row,
                jnp.broadcast_to(fvec_ref[...], (P, PA)),
                0.0,
            )

        ut_blk = ut_ref[...]
        fh = (ut_blk >= spc_ref[...]).astype(jnp.float32)
        tt = fh * ut_blk
        ft_ref[...] = fh
        y_ref[...] = jax.lax.dot_general(
            e_ref[...], tt,
            (((0,), (0,)), ((), ())),
            precision=jax.lax.Precision.DEFAULT,
            preferred_element_type=jnp.float32,
        )

    return pl.pallas_call(
        tc_body,
        grid=(N // BN,),
        in_specs=[
            pl.BlockSpec((P, BN), lambda i: (0, i)),
            pl.BlockSpec((1, PA), lambda i: (0, 0)),
            pl.BlockSpec((P, 1), lambda i: (0, 0)),
        ],
        out_specs=[
            pl.BlockSpec((P, BN), lambda i: (0, i)),
            pl.BlockSpec((PA, BN), lambda i: (0, i)),
        ],
        out_shape=[
            jax.ShapeDtypeStruct((P, N), jnp.float32),
            jax.ShapeDtypeStruct((PA, N), jnp.float32),
        ],
        scratch_shapes=[pltpu.VMEM((P, PA), jnp.float32)],
    )(ut, fvec, spc)


def kernel(input_ids, W, f, SP):
    P, V = W.shape
    A = f.shape[1]
    tok = input_ids.reshape(-1)
    N = tok.shape[0]

    info = plsc.get_sparse_core_info()
    ut = _sc_gather(W, tok, info.num_cores, info.num_subcores, info.num_lanes)

    fvec = f.reshape(1, P * A)
    spc = SP.reshape(P, 1)
    ft, z = _tc_expand(ut, fvec, spc, BN=1024)
    y = z.reshape(P, A, N).transpose(2, 0, 1)
    return ut.T, ft.T, y
